# baseline (speedup 1.0000x reference)
import functools

import jax
import jax.numpy as jnp
from jax import lax
from jax.experimental import pallas as pl
from jax.experimental.pallas import tpu as pltpu
from jax.experimental.pallas import tpu_sc as plsc

NX = 432
NY = 496
NP = NX * NY
NF = 64
NB = 4
NPILLARS = NB * 25000

NC = 2
NS = 16
NW = NC * NS
WPB = NW // NB

PIX_BIG = 26880
PIX_LAST = NP - 7 * PIX_BIG

CHUNK = 384
NCH_BIG = PIX_BIG // CHUNK
NCH_LAST = PIX_LAST // CHUNK

PCHUNK = 4000
NPCHUNK = NPILLARS // PCHUNK


def _body(coords_hbm, feat_hbm, out_hbm, coords_v, inv_v, idx_v, rows_v,
          block_v, sem):
    lanes = lax.iota(jnp.int32, 16)
    wid = lax.axis_index("c") * NS + lax.axis_index("s")
    batch = wid // WPB
    k = wid % WPB
    start = k * PIX_BIG
    last = k == (WPB - 1)
    my_size = jnp.where(last, PIX_LAST, PIX_BIG)
    nchunks = jnp.where(last, NCH_LAST, NCH_BIG)
    base = batch * NP + start

    def init_body(i, _):
        inv_v[pl.ds(i * 16, 16)] = jnp.full((16,), -1, jnp.int32)
        return None
    lax.fori_loop(0, PIX_BIG // 16, init_body, None)

    def scan_chunk(c, _):
        pltpu.sync_copy(coords_hbm.at[pl.ds(c * PCHUNK, PCHUNK), :], coords_v)

        def scan_group(g, _):
            rows = g * 16 + lanes
            bcol = plsc.load_gather(coords_v, [rows, jnp.zeros((16,), jnp.int32)])
            y = plsc.load_gather(coords_v, [rows, jnp.full((16,), 2, jnp.int32)])
            x = plsc.load_gather(coords_v, [rows, jnp.full((16,), 3, jnp.int32)])
            gidx = bcol * NP + y * NX + x
            local = gidx - base
            m = (local >= 0) & (local < my_size)
            safe = jnp.clip(local, 0, PIX_BIG - 1)
            pid = c * PCHUNK + rows
            plsc.store_scatter(inv_v, [safe], pid, mask=m)
            return None
        lax.fori_loop(0, PCHUNK // 16, scan_group, None)
        return None
    lax.fori_loop(0, NPCHUNK, scan_chunk, None)

    def do_chunk(t, _):
        p_lo = t * CHUNK

        def mk_idx(g, _):
            iv = inv_v[pl.ds(p_lo + g * 16, 16)]
            idx_v[pl.ds(g * 16, 16)] = jnp.maximum(iv, 0)
            return None
        lax.fori_loop(0, CHUNK // 16, mk_idx, None)

        for off in range(0, CHUNK, 128):
            pltpu.async_copy(
                feat_hbm.at[idx_v.at[pl.ds(off, 128)]],
                rows_v.at[pl.ds(off, 128), :], sem).wait()

        def transpose_group(g, _):
            j0 = g * 16
            iv = inv_v[pl.ds(p_lo + j0, 16)]
            m = iv >= 0
            rows = j0 + lanes
            for f in range(NF):
                col = plsc.load_gather(rows_v, [rows, jnp.full((16,), f, jnp.int32)])
                block_v[f, pl.ds(j0, 16)] = jnp.where(m, col, 0.0)
            return None
        lax.fori_loop(0, CHUNK // 16, transpose_group, None)

        pltpu.sync_copy(block_v, out_hbm.at[batch, :, pl.ds(start + p_lo, CHUNK)])
        return None
    lax.fori_loop(0, nchunks, do_chunk, None)


_scatter = functools.partial(
    pl.kernel,
    out_type=jax.ShapeDtypeStruct((NB, NF, NP), jnp.float32),
    mesh=plsc.VectorSubcoreMesh(core_axis_name="c", subcore_axis_name="s"),
    scratch_types=[
        pltpu.VMEM((PCHUNK, 4), jnp.int32),
        pltpu.VMEM((PIX_BIG,), jnp.int32),
        pltpu.VMEM((CHUNK,), jnp.int32),
        pltpu.VMEM((CHUNK, NF), jnp.float32),
        pltpu.VMEM((NF, CHUNK), jnp.float32),
        pltpu.SemaphoreType.DMA,
    ],
    compiler_params=pltpu.CompilerParams(
        needs_layout_passes=False, use_tc_tiling_on_sc=False),
)(_body)


@jax.jit
def kernel(voxel_coords, pillar_features):
    out = _scatter(voxel_coords.astype(jnp.int32), pillar_features)
    return out.reshape(NB, NF, NY, NX)


# speedup vs baseline: 10.4311x; 10.4311x over previous
import functools

import jax
import jax.numpy as jnp
from jax import lax
from jax.experimental import pallas as pl
from jax.experimental.pallas import tpu as pltpu
from jax.experimental.pallas import tpu_sc as plsc

NX = 432
NY = 496
NP = NX * NY
NF = 64
NB = 4
PPB = 25000
NPILLARS = NB * PPB

NS = 16
WPB = 8

PIX_BIG = 26880
PIX_LAST = NP - 7 * PIX_BIG

CHUNK = 256
BLKW = CHUNK + 1
NCH_BIG = PIX_BIG // CHUNK
NCH_LAST = PIX_LAST // CHUNK

PCHUNK = 4992
PTAIL = PPB - 5 * PCHUNK


def _body(y_hbm, x_hbm, feat_hbm, out_hbm,
          y_v, x_v, inv_v,
          cidx0, cidx1, cj0, cj1, rows0, rows1, blk0, blk1,
          ysem, gsem0, gsem1, osem0, osem1):
    lanes = lax.iota(jnp.int32, 16)
    i32 = jnp.int32
    wid = lax.axis_index("c") * NS + lax.axis_index("s")
    batch = wid // WPB
    k = wid % WPB
    start = k * PIX_BIG
    last = k == (WPB - 1)
    my_size = jnp.where(last, PIX_LAST, PIX_BIG)
    nchunks = jnp.where(last, NCH_LAST, NCH_BIG)
    prow0 = batch * PPB

    def init_body(i, _):
        inv_v[pl.ds(i * 16, 16)] = jnp.full((16,), -1, i32)
        return None
    lax.fori_loop(0, PIX_BIG // 16, init_body, None)

    def scan_groups(base_row, g, extra_mask_len):
        yv = y_v[pl.ds(g * 16, 16)]
        xv = x_v[pl.ds(g * 16, 16)]
        pix = yv * NX + xv
        local = pix - start
        m = (local >= 0) & (local < my_size)
        if extra_mask_len is not None:
            m = m & (g * 16 + lanes < extra_mask_len)
        safe = jnp.clip(local, 0, PIX_BIG - 1)
        pid = base_row + g * 16 + lanes
        plsc.store_scatter(inv_v, [safe], pid, mask=m)

    def scan_chunk(c, _):
        off = prow0 + c * PCHUNK
        pltpu.async_copy(y_hbm.at[pl.ds(off, PCHUNK)], y_v, ysem)
        pltpu.async_copy(x_hbm.at[pl.ds(off, PCHUNK)], x_v, ysem)
        pltpu.make_async_copy(y_hbm.at[pl.ds(off, PCHUNK)], y_v, ysem).wait()
        pltpu.make_async_copy(x_hbm.at[pl.ds(off, PCHUNK)], x_v, ysem).wait()

        def g_body(g, _):
            scan_groups(off, g, None)
            return None
        lax.fori_loop(0, PCHUNK // 16, g_body, None)
        return None
    lax.fori_loop(0, 5, scan_chunk, None)

    toff = prow0 + 5 * PCHUNK
    pltpu.async_copy(y_hbm.at[pl.ds(toff, 48)], y_v.at[pl.ds(0, 48)], ysem)
    pltpu.async_copy(x_hbm.at[pl.ds(toff, 48)], x_v.at[pl.ds(0, 48)], ysem)
    pltpu.make_async_copy(y_hbm.at[pl.ds(toff, 48)], y_v.at[pl.ds(0, 48)], ysem).wait()
    pltpu.make_async_copy(x_hbm.at[pl.ds(toff, 48)], x_v.at[pl.ds(0, 48)], ysem).wait()
    for g in range(3):
        scan_groups(toff, g, PTAIL)

    def do_chunk(t, cidx_b, cj_b, rows_b, blk_b, gsem_b, osem_b):
        p_lo = t * CHUNK

        def compact(g, off):
            iv = inv_v[pl.ds(p_lo + g * 16, 16)]
            m = iv >= 0
            mi = m.astype(i32)
            incl = plsc.cumsum(mi)
            pos = off + incl - 1
            plsc.store_scatter(cidx_b, [pos], iv, mask=m)
            plsc.store_scatter(cj_b, [pos], g * 16 + lanes, mask=m)
            return off + jnp.sum(mi)
        nv = lax.fori_loop(0, CHUNK // 16, compact, jnp.int32(0))

        plsc.store_scatter(cidx_b, [nv + lanes], prow0 + lanes,
                           mask=(nv + lanes) < CHUNK)
        ngd = (nv + 15) // 16

        def fire(d, _):
            pltpu.async_copy(feat_hbm.at[cidx_b.at[pl.ds(d * 16, 16)]],
                             rows_b.at[pl.ds(d * 16, 16), :], gsem_b)
            return None
        lax.fori_loop(0, ngd, fire, None)

        @pl.when(t >= 2)
        def _():
            pltpu.make_async_copy(
                blk_b.at[:, pl.ds(0, CHUNK)],
                out_hbm.at[batch, :, pl.ds(start + p_lo, CHUNK)],
                osem_b).wait()

        def zero_row(f, _):
            for g2 in range(CHUNK // 16):
                blk_b[f, pl.ds(g2 * 16, 16)] = jnp.zeros((16,), jnp.float32)
            return None
        lax.fori_loop(0, NF, zero_row, None)

        def drain(d, _):
            pltpu.make_async_copy(feat_hbm.at[cidx_b.at[pl.ds(d * 16, 16)]],
                                  rows_b.at[pl.ds(d * 16, 16), :], gsem_b).wait()
            return None
        lax.fori_loop(0, ngd, drain, None)

        def sgroup(t2, _):
            for l in range(16):
                slot = t2 * 16 + l
                mv = jnp.broadcast_to(slot < nv, (16,))
                jb = plsc.load_gather(cj_b, [jnp.full((16,), slot, i32)])
                for q in range(4):
                    rv = rows_b[slot, pl.ds(q * 16, 16)]
                    plsc.store_scatter(blk_b, [q * 16 + lanes, jb], rv, mask=mv)
            return None
        lax.fori_loop(0, ngd, sgroup, None)

        pltpu.async_copy(blk_b.at[:, pl.ds(0, CHUNK)],
                         out_hbm.at[batch, :, pl.ds(start + p_lo, CHUNK)],
                         osem_b)

    def outer(t2, _):
        t = t2 * 2

        @pl.when(t < nchunks)
        def _():
            do_chunk(t, cidx0, cj0, rows0, blk0, gsem0, osem0)

        @pl.when(t + 1 < nchunks)
        def _():
            do_chunk(t + 1, cidx1, cj1, rows1, blk1, gsem1, osem1)
        return None
    lax.fori_loop(0, (NCH_BIG + 1) // 2, outer, None)

    pltpu.make_async_copy(blk0.at[:, pl.ds(0, CHUNK)],
                          out_hbm.at[batch, :, pl.ds(start, CHUNK)], osem0).wait()
    pltpu.make_async_copy(blk1.at[:, pl.ds(0, CHUNK)],
                          out_hbm.at[batch, :, pl.ds(start, CHUNK)], osem1).wait()


_scatter = functools.partial(
    pl.kernel,
    out_type=jax.ShapeDtypeStruct((NB, NF, NP), jnp.float32),
    mesh=plsc.VectorSubcoreMesh(core_axis_name="c", subcore_axis_name="s"),
    scratch_types=[
        pltpu.VMEM((PCHUNK,), jnp.int32),
        pltpu.VMEM((PCHUNK,), jnp.int32),
        pltpu.VMEM((PIX_BIG,), jnp.int32),
        pltpu.VMEM((CHUNK,), jnp.int32),
        pltpu.VMEM((CHUNK,), jnp.int32),
        pltpu.VMEM((CHUNK,), jnp.int32),
        pltpu.VMEM((CHUNK,), jnp.int32),
        pltpu.VMEM((CHUNK, NF), jnp.float32),
        pltpu.VMEM((CHUNK, NF), jnp.float32),
        pltpu.VMEM((NF, BLKW), jnp.float32),
        pltpu.VMEM((NF, BLKW), jnp.float32),
        pltpu.SemaphoreType.DMA,
        pltpu.SemaphoreType.DMA,
        pltpu.SemaphoreType.DMA,
        pltpu.SemaphoreType.DMA,
        pltpu.SemaphoreType.DMA,
    ],
    compiler_params=pltpu.CompilerParams(
        needs_layout_passes=False, use_tc_tiling_on_sc=False),
)(_body)


@jax.jit
def kernel(voxel_coords, pillar_features):
    vc = voxel_coords.astype(jnp.int32)
    pad = jnp.zeros((128,), jnp.int32)
    y = jnp.concatenate([vc[:, 2], pad])
    x = jnp.concatenate([vc[:, 3], pad])
    out = _scatter(y, x, pillar_features)
    return out.reshape(NB, NF, NY, NX)
